# baseline (speedup 1.0000x reference)
import functools

import jax
import jax.numpy as jnp
from jax import lax
from jax.experimental import pallas as pl
from jax.experimental.pallas import tpu as pltpu
from jax.experimental.pallas import tpu_sc as plsc

B = 4096
L = 200
E = 64
EPS = 1e-5

NC = 2
NS = 16
NW = NC * NS
BPW = B // NW
CH = 100
NCH = BPW * 2
NBUF = 4

_LANES = 16
_Q = E // _LANES


def _bcast(x):
    return jnp.broadcast_to(x, (_LANES,))


def _newton_rsqrt(v):
    i = lax.bitcast_convert_type(v, jnp.int32)
    i = jnp.int32(0x5F3759DF) - (i >> 1)
    y = lax.bitcast_convert_type(i, jnp.float32)
    half = v * 0.5
    for _ in range(2):
        y = y * (1.5 - half * y * y)
    return y


def _body(seq_hbm, mkts_hbm, tok_hbm, mkt_tbl_hbm, gamma_hbm, beta_hbm,
          out_hbm, idx_v, mkt_idx_v, gsem, wsem, msem):
    pl.run_scoped(
        functools.partial(
            _scoped_body, seq_hbm, mkts_hbm, tok_hbm, mkt_tbl_hbm, gamma_hbm,
            beta_hbm, out_hbm, idx_v, mkt_idx_v, gsem, wsem, msem),
        pltpu.VMEM((BPW, E), jnp.float32),
        pltpu.VMEM((E,), jnp.float32),
        pltpu.VMEM((E,), jnp.float32),
        pltpu.VMEM((NBUF, CH, E), jnp.float32),
        pltpu.VMEM((NBUF, CH, E), jnp.float32),
    )


def _scoped_body(seq_hbm, mkts_hbm, tok_hbm, mkt_tbl_hbm, gamma_hbm, beta_hbm,
                 out_hbm, idx_v, mkt_idx_v, gsem, wsem, msem,
                 mkt_rows_v, g_v, b_v, rows_v, outs_v):
    wid = lax.axis_index("s") * NC + lax.axis_index("c")
    b0 = wid * BPW

    pltpu.sync_copy(seq_hbm.at[wid], idx_v)
    pltpu.sync_copy(mkts_hbm.at[wid], mkt_idx_v)
    pltpu.sync_copy(gamma_hbm, g_v)
    pltpu.sync_copy(beta_hbm, b_v)
    pltpu.async_copy(mkt_tbl_hbm.at[mkt_idx_v], mkt_rows_v, msem).wait()

    lane = lax.iota(jnp.int32, _LANES)
    lq = [lane + q * _LANES for q in range(_Q)]
    gq = [plsc.load_gather(g_v, [lq[q]]) for q in range(_Q)]
    bq = [plsc.load_gather(b_v, [lq[q]]) for q in range(_Q)]

    def _idx_row(c):
        half = c - 2 * (c // 2)
        return idx_v.at[c // 2, half]

    def _out_slice(c):
        half = c - 2 * (c // 2)
        return out_hbm.at[b0 + c // 2, pl.ds(half * CH, CH)]

    for k in range(NBUF):
        pltpu.async_copy(tok_hbm.at[_idx_row(k)], rows_v.at[k], gsem.at[k])

    @pl.loop(0, NCH // NBUF)
    def _chunks(i):
        for k in range(NBUF):
            c = i * NBUF + k

            @pl.when(i > 0)
            def _():
                pltpu.make_async_copy(
                    outs_v.at[k], _out_slice(c - NBUF), wsem.at[k]
                ).wait()

            pltpu.make_async_copy(
                tok_hbm.at[_idx_row(c)], rows_v.at[k], gsem.at[k]
            ).wait()

            b_vec = _bcast(c // 2).astype(jnp.int32)
            mq = [plsc.load_gather(mkt_rows_v, [b_vec, lq[q]])
                  for q in range(_Q)]
            k_vec = jnp.full((_LANES,), k, jnp.int32)

            @pl.loop(0, CH, unroll=4)
            def _rows(r):
                r_vec = _bcast(r).astype(jnp.int32)
                x = [plsc.load_gather(rows_v, [k_vec, r_vec, lq[q]]) + mq[q]
                     for q in range(_Q)]
                s = (x[0] + x[1]) + (x[2] + x[3])
                s2 = (x[0] * x[0] + x[1] * x[1]) + (x[2] * x[2] + x[3] * x[3])
                mean = _bcast(jnp.sum(s)) * (1.0 / E)
                ex2 = _bcast(jnp.sum(s2)) * (1.0 / E)
                var = ex2 - mean * mean
                r_ = _newton_rsqrt(var + EPS)
                for q in range(_Q):
                    a = gq[q] * r_
                    plsc.store_scatter(outs_v, [k_vec, r_vec, lq[q]],
                                       (x[q] - mean) * a + bq[q])

            pltpu.async_copy(outs_v.at[k], _out_slice(c), wsem.at[k])

            @pl.when(c + NBUF < NCH)
            def _():
                pltpu.async_copy(
                    tok_hbm.at[_idx_row(c + NBUF)], rows_v.at[k], gsem.at[k])

    for k in range(NBUF):
        pltpu.make_async_copy(
            outs_v.at[k], _out_slice(NCH - NBUF + k), wsem.at[k]
        ).wait()


_sc_call = pl.kernel(
    _body,
    out_type=jax.ShapeDtypeStruct((B, L, E), jnp.float32),
    mesh=plsc.VectorSubcoreMesh(core_axis_name="c", subcore_axis_name="s"),
    compiler_params=pltpu.CompilerParams(
        needs_layout_passes=False, use_tc_tiling_on_sc=False),
    scratch_types=[
        pltpu.VMEM((BPW, 2, CH), jnp.int32),
        pltpu.VMEM((BPW,), jnp.int32),
        pltpu.SemaphoreType.DMA((NBUF,)),
        pltpu.SemaphoreType.DMA((NBUF,)),
        pltpu.SemaphoreType.DMA,
    ],
)


@jax.jit
def kernel(sequence, mkts, token_table, market_table, gamma, beta):
    seq_r = sequence.astype(jnp.int32).reshape(NW, BPW, 2, CH)
    mkts_r = mkts.astype(jnp.int32).reshape(NW, BPW)
    return _sc_call(seq_r, mkts_r, token_table, market_table, gamma, beta)
